# baseline (speedup 1.0000x reference)
import functools

import jax
import jax.numpy as jnp
import numpy as np
from jax import lax
from jax.experimental import pallas as pl
from jax.experimental.pallas import tpu as pltpu
from jax.experimental.pallas import tpu_sc as plsc

L = 16
F = 2
T = 524288
BASE = 16
SCALE = 1.3819
RES = [int(np.floor(BASE * (SCALE ** l))) for l in range(L)]
P1 = np.uint32(2654435761)
P2 = np.uint32(805459861)
MASK = np.uint32(T - 1)

NC = 2
NS = 16
NW = NC * NS

N = 1048576
C = 1024
G = C // 16
PPW = N // NW
NCHUNK = PPW // C


def _enc_body(x_hbm, tbl_hbm, out_hbm, xv, fxv, fyv, fzv, idxv, rowsv, encv,
              sem0, sem1):
    wid = lax.axis_index("s") * NC + lax.axis_index("c")
    lane = lax.iota(jnp.int32, 16)
    sems = (sem0, sem1)

    def chunk_body(ci, carry):
        base = wid * PPW + ci * C
        for d in range(3):
            pltpu.sync_copy(x_hbm.at[pl.ds(d * N + base, C)], xv.at[d])

        def run_a(l, bf):
            res_half = float(RES[l]) * 0.5
            lOFF = l * (2 * T)

            def group_a(g, carry_a):
                p0 = g * 16
                xr = xv[0, pl.ds(p0, 16)]
                yr = xv[1, pl.ds(p0, 16)]
                zr = xv[2, pl.ds(p0, 16)]
                px = (xr + 1.0) * res_half
                py = (yr + 1.0) * res_half
                pz = (zr + 1.0) * res_half
                ix = px.astype(jnp.int32)
                iy = py.astype(jnp.int32)
                iz = pz.astype(jnp.int32)
                fxv[bf, pl.ds(p0, 16)] = px - ix.astype(jnp.float32)
                fyv[bf, pl.ds(p0, 16)] = py - iy.astype(jnp.float32)
                fzv[bf, pl.ds(p0, 16)] = pz - iz.astype(jnp.float32)
                a0 = ix.astype(jnp.uint32)
                a1 = a0 + jnp.uint32(1)
                b0 = iy.astype(jnp.uint32) * P1
                b1 = b0 + P1
                c0 = iz.astype(jnp.uint32) * P2
                c1 = c0 + P2
                cc = 0
                for av in (a0, a1):
                    for bv in (b0, b1):
                        for cv in (c0, c1):
                            h = (av ^ bv ^ cv) & MASK
                            off = (((h >> jnp.uint32(7)) << jnp.uint32(8))
                                   | (h & jnp.uint32(127))).astype(jnp.int32)
                            off = off + lOFF
                            idxv[bf, pl.ds(2 * cc * C + p0, 16)] = off
                            idxv[bf, pl.ds((2 * cc + 1) * C + p0, 16)] = off + 128
                            cc += 1
                return carry_a

            lax.fori_loop(0, G, group_a, 0)
            return [pltpu.async_copy(
                tbl_hbm.at[idxv.at[bf, pl.ds(q * 4 * C, 4 * C)]],
                rowsv.at[bf, pl.ds(q * 4 * C, 4 * C)],
                sems[bf]) for q in range(4)]

        def run_b(l, bf):
            def group_b(g, carry_b):
                p0 = g * 16
                fx = fxv[bf, pl.ds(p0, 16)]
                fy = fyv[bf, pl.ds(p0, 16)]
                fz = fzv[bf, pl.ds(p0, 16)]
                for f in (0, 1):
                    v = [rowsv[bf, pl.ds((2 * c + f) * C + p0, 16)]
                         for c in range(8)]
                    m00 = v[0] + fz * (v[1] - v[0])
                    m01 = v[2] + fz * (v[3] - v[2])
                    m10 = v[4] + fz * (v[5] - v[4])
                    m11 = v[6] + fz * (v[7] - v[6])
                    n0 = m00 + fy * (m01 - m00)
                    n1 = m10 + fy * (m11 - m10)
                    encv[2 * l + f, pl.ds(p0, 16)] = n0 + fx * (n1 - n0)
                return carry_b

            lax.fori_loop(0, G, group_b, 0)

        pending = run_a(0, 0)
        for l in range(1, L):
            nxt = run_a(l, l % 2)
            for p in pending:
                p.wait()
            run_b(l - 1, (l - 1) % 2)
            pending = nxt
        for p in pending:
            p.wait()
        run_b(L - 1, (L - 1) % 2)

        for f2 in range(2 * L):
            pltpu.sync_copy(encv.at[f2], out_hbm.at[pl.ds(f2 * N + base, C)])
        return carry

    lax.fori_loop(0, NCHUNK, chunk_body, 0)


def _encode_sc(x_flat, tbl_flat):
    mesh = plsc.VectorSubcoreMesh(core_axis_name="c", subcore_axis_name="s")
    k = functools.partial(
        pl.kernel,
        mesh=mesh,
        out_type=jax.ShapeDtypeStruct((2 * L * N,), jnp.float32),
        scratch_types=[
            pltpu.VMEM((3, C), jnp.float32),
            pltpu.VMEM((2, C), jnp.float32),
            pltpu.VMEM((2, C), jnp.float32),
            pltpu.VMEM((2, C), jnp.float32),
            pltpu.VMEM((2, 16 * C), jnp.int32),
            pltpu.VMEM((2, 16 * C), jnp.float32),
            pltpu.VMEM((2 * L, C), jnp.float32),
            pltpu.SemaphoreType.DMA,
            pltpu.SemaphoreType.DMA,
        ],
        compiler_params=pltpu.CompilerParams(use_tc_tiling_on_sc=False,
                                             needs_layout_passes=False),
    )(_enc_body)
    return k(x_flat, tbl_flat)


def _mlp_body(enc_ref, w1_ref, w2_ref, w3_ref, out_ref):
    hp = jax.lax.Precision.HIGHEST
    enc = enc_ref[...]
    h1 = jax.lax.dot_general(w1_ref[...], enc, (((0,), (0,)), ((), ())),
                             precision=hp, preferred_element_type=jnp.float32)
    h1 = jnp.maximum(h1, 0.0)
    h2 = jax.lax.dot_general(w2_ref[...], h1, (((0,), (0,)), ((), ())),
                             precision=hp, preferred_element_type=jnp.float32)
    h2 = jnp.maximum(h2, 0.0)
    out_ref[...] = jax.lax.dot_general(w3_ref[...], h2, (((0,), (0,)), ((), ())),
                                       precision=hp,
                                       preferred_element_type=jnp.float32)


def _mlp(enc_t, W1, W2, W3):
    n = enc_t.shape[1]
    bb = 8192
    grid = (n // bb,)
    return pl.pallas_call(
        _mlp_body,
        grid=grid,
        in_specs=[
            pl.BlockSpec((2 * L, bb), lambda i: (0, i)),
            pl.BlockSpec((32, 64), lambda i: (0, 0)),
            pl.BlockSpec((64, 64), lambda i: (0, 0)),
            pl.BlockSpec((64, 1), lambda i: (0, 0)),
        ],
        out_specs=pl.BlockSpec((1, bb), lambda i: (0, i)),
        out_shape=jax.ShapeDtypeStruct((1, n), jnp.float32),
    )(enc_t, W1, W2, W3)


def kernel(x, tables, W1, W2, W3):
    n = x.shape[0]
    x_flat = jnp.transpose(x).reshape(3 * n)
    tbl_flat = (tables.reshape(L, T // 128, 128, 2)
                .transpose(0, 1, 3, 2)
                .reshape(L * T * 2))
    enc_flat = _encode_sc(x_flat, tbl_flat)
    enc_t = enc_flat.reshape(2 * L, n)
    out_t = _mlp(enc_t, W1, W2, W3)
    return out_t.reshape(n, 1)


# speedup vs baseline: 1.2580x; 1.2580x over previous
import functools

import jax
import jax.numpy as jnp
import numpy as np
from jax import lax
from jax.experimental import pallas as pl
from jax.experimental.pallas import tpu as pltpu
from jax.experimental.pallas import tpu_sc as plsc

L = 16
F = 2
T = 524288
BASE = 16
SCALE = 1.3819
RES = [int(np.floor(BASE * (SCALE ** l))) for l in range(L)]
P1 = np.uint32(2654435761)
P2 = np.uint32(805459861)
MASK = np.uint32(T - 1)

NC = 2
NS = 16
NW = NC * NS

N = 1048576
C = 1024
G = C // 16
PPW = N // NW
NCHUNK = PPW // C


def _enc_body(x_hbm, tbl_hbm, out_hbm, xv, fxv, fyv, fzv, idxv, rowsv, encv,
              sem0, sem1):
    wid = lax.axis_index("s") * NC + lax.axis_index("c")
    lane = lax.iota(jnp.int32, 16)
    sems = (sem0, sem1)

    def chunk_body(ci, carry):
        base = wid * PPW + ci * C
        for d in range(3):
            pltpu.sync_copy(x_hbm.at[pl.ds(d * N + base, C)], xv.at[d])

        def run_a(l, bf):
            res_half = float(RES[l]) * 0.5
            lOFF = l * (2 * T)

            def group_a(g, carry_a):
                p0 = g * 16
                xr = xv[0, pl.ds(p0, 16)]
                yr = xv[1, pl.ds(p0, 16)]
                zr = xv[2, pl.ds(p0, 16)]
                px = (xr + 1.0) * res_half
                py = (yr + 1.0) * res_half
                pz = (zr + 1.0) * res_half
                ix = px.astype(jnp.int32)
                iy = py.astype(jnp.int32)
                iz = pz.astype(jnp.int32)
                fxv[bf, pl.ds(p0, 16)] = px - ix.astype(jnp.float32)
                fyv[bf, pl.ds(p0, 16)] = py - iy.astype(jnp.float32)
                fzv[bf, pl.ds(p0, 16)] = pz - iz.astype(jnp.float32)
                a0 = ix.astype(jnp.uint32)
                a1 = a0 + jnp.uint32(1)
                b0 = iy.astype(jnp.uint32) * P1
                b1 = b0 + P1
                c0 = iz.astype(jnp.uint32) * P2
                c1 = c0 + P2
                cc = 0
                for av in (a0, a1):
                    for bv in (b0, b1):
                        for cv in (c0, c1):
                            h = (av ^ bv ^ cv) & MASK
                            off = (((h >> jnp.uint32(7)) << jnp.uint32(8))
                                   | (h & jnp.uint32(127))).astype(jnp.int32)
                            off = off + lOFF
                            idxv[bf, pl.ds(2 * cc * C + p0, 16)] = off
                            idxv[bf, pl.ds((2 * cc + 1) * C + p0, 16)] = off + 128
                            cc += 1
                return carry_a

            lax.fori_loop(0, G, group_a, 0)
            return [pltpu.async_copy(
                tbl_hbm.at[idxv.at[bf, pl.ds(q * 4 * C, 4 * C)]],
                rowsv.at[bf, pl.ds(q * 4 * C, 4 * C)],
                sems[bf]) for q in range(4)]

        def run_b(l, bf):
            def group_b(g, carry_b):
                p0 = g * 16
                fx = fxv[bf, pl.ds(p0, 16)]
                fy = fyv[bf, pl.ds(p0, 16)]
                fz = fzv[bf, pl.ds(p0, 16)]
                for f in (0, 1):
                    v = [rowsv[bf, pl.ds((2 * c + f) * C + p0, 16)]
                         for c in range(8)]
                    m00 = v[0] + fz * (v[1] - v[0])
                    m01 = v[2] + fz * (v[3] - v[2])
                    m10 = v[4] + fz * (v[5] - v[4])
                    m11 = v[6] + fz * (v[7] - v[6])
                    n0 = m00 + fy * (m01 - m00)
                    n1 = m10 + fy * (m11 - m10)
                    f2 = 2 * l + f
                    off = ((f2 // 8) * 8192 + (f2 % 8) * 128
                           + (p0 >> 7) * 1024 + (p0 & 127))
                    encv[pl.ds(off, 16)] = n0 + fx * (n1 - n0)
                return carry_b

            lax.fori_loop(0, G, group_b, 0)

        pending = run_a(0, 0)
        for l in range(1, L):
            nxt = run_a(l, l % 2)
            for p in pending:
                p.wait()
            run_b(l - 1, (l - 1) % 2)
            pending = nxt
        for p in pending:
            p.wait()
        run_b(L - 1, (L - 1) % 2)

        for ft in range(4):
            pltpu.sync_copy(
                encv.at[pl.ds(ft * 8192, 8192)],
                out_hbm.at[pl.ds(ft * (8 * N) + base * 8, 8 * C)])
        return carry

    lax.fori_loop(0, NCHUNK, chunk_body, 0)


def _encode_sc(x_flat, tbl_flat):
    mesh = plsc.VectorSubcoreMesh(core_axis_name="c", subcore_axis_name="s")
    k = functools.partial(
        pl.kernel,
        mesh=mesh,
        out_type=jax.ShapeDtypeStruct((2 * L * N,), jnp.float32),
        scratch_types=[
            pltpu.VMEM((3, C), jnp.float32),
            pltpu.VMEM((2, C), jnp.float32),
            pltpu.VMEM((2, C), jnp.float32),
            pltpu.VMEM((2, C), jnp.float32),
            pltpu.VMEM((2, 16 * C), jnp.int32),
            pltpu.VMEM((2, 16 * C), jnp.float32),
            pltpu.VMEM((2 * L * C,), jnp.float32),
            pltpu.SemaphoreType.DMA,
            pltpu.SemaphoreType.DMA,
        ],
        compiler_params=pltpu.CompilerParams(use_tc_tiling_on_sc=False,
                                             needs_layout_passes=False),
    )(_enc_body)
    return k(x_flat, tbl_flat)


def _mlp_body(enc_ref, w1_ref, w2_ref, w3_ref, out_ref):
    hp = jax.lax.Precision.HIGHEST
    enc = enc_ref[...]
    h1 = jax.lax.dot_general(w1_ref[...], enc, (((0,), (0,)), ((), ())),
                             precision=hp, preferred_element_type=jnp.float32)
    h1 = jnp.maximum(h1, 0.0)
    h2 = jax.lax.dot_general(w2_ref[...], h1, (((0,), (0,)), ((), ())),
                             precision=hp, preferred_element_type=jnp.float32)
    h2 = jnp.maximum(h2, 0.0)
    out_ref[...] = jax.lax.dot_general(w3_ref[...], h2, (((0,), (0,)), ((), ())),
                                       precision=hp,
                                       preferred_element_type=jnp.float32)


def _mlp(enc_t, W1, W2, W3):
    n = enc_t.shape[1]
    bb = 8192
    grid = (n // bb,)
    return pl.pallas_call(
        _mlp_body,
        grid=grid,
        in_specs=[
            pl.BlockSpec((2 * L, bb), lambda i: (0, i)),
            pl.BlockSpec((32, 64), lambda i: (0, 0)),
            pl.BlockSpec((64, 64), lambda i: (0, 0)),
            pl.BlockSpec((64, 1), lambda i: (0, 0)),
        ],
        out_specs=pl.BlockSpec((1, bb), lambda i: (0, i)),
        out_shape=jax.ShapeDtypeStruct((1, n), jnp.float32),
    )(enc_t, W1, W2, W3)


def kernel(x, tables, W1, W2, W3):
    n = x.shape[0]
    x_flat = jnp.transpose(x).reshape(3 * n)
    tbl_flat = (tables.reshape(L, T // 128, 128, 2)
                .transpose(0, 1, 3, 2)
                .reshape(L * T * 2))
    enc_flat = _encode_sc(x_flat, tbl_flat)
    enc_t = (enc_flat.reshape(4, n // 128, 8, 128)
             .transpose(0, 2, 1, 3)
             .reshape(2 * L, n))
    out_t = _mlp(enc_t, W1, W2, W3)
    return out_t.reshape(n, 1)


# speedup vs baseline: 2.2859x; 1.8170x over previous
import functools

import jax
import jax.numpy as jnp
import numpy as np
from jax import lax
from jax.experimental import pallas as pl
from jax.experimental.pallas import tpu as pltpu
from jax.experimental.pallas import tpu_sc as plsc

L = 16
F = 2
T = 524288
T4 = T // 4
BASE = 16
SCALE = 1.3819
RES = [int(np.floor(BASE * (SCALE ** l))) for l in range(L)]
P1 = np.uint32(2654435761)
P2 = np.uint32(805459861)
MASK = np.uint32(T - 1)

NC = 2
NS = 16
NW = NC * NS

N = 1048576
C = 512
G = C // 16
PPW = N // NW
NCHUNK = PPW // C

_CP = pltpu.CompilerParams(use_tc_tiling_on_sc=False,
                           needs_layout_passes=False)


PK_BLK = 8192
PK_SLAB = (2 * T * L) // NW
PK_STEPS = PK_BLK // 256


def _pack_body(tbl_hbm, out_hbm, bufv, pkv, sem):
    wid = lax.axis_index("s") * NC + lax.axis_index("c")
    lane = lax.iota(jnp.int32, 16)
    row_pat = lane >> 2
    col_pat = (lane & 3) * 2
    slab0 = wid * PK_SLAB

    def block_body(kb, carry):
        e0 = slab0 + kb * PK_BLK
        pltpu.sync_copy(tbl_hbm.at[pl.ds(e0, PK_BLK)], bufv)

        def unit_body(s, c2):
            u = s >> 3
            k0 = (s & 7) * 16
            src = u * 256 + k0
            f0 = bufv[pl.ds(src, 16)]
            f1 = bufv[pl.ds(src + 128, 16)]
            row = u * 32 + (k0 >> 2) + row_pat
            plsc.store_scatter(pkv, [row, col_pat], f0)
            plsc.store_scatter(pkv, [row, col_pat + 1], f1)
            return c2

        lax.fori_loop(0, PK_STEPS * 8, unit_body, 0)
        pltpu.sync_copy(pkv, out_hbm.at[pl.ds(e0 // 8, PK_BLK // 8), :])
        return carry

    lax.fori_loop(0, PK_SLAB // PK_BLK, block_body, 0)


def _pack_sc(tbl_flat):
    mesh = plsc.VectorSubcoreMesh(core_axis_name="c", subcore_axis_name="s")
    k = functools.partial(
        pl.kernel,
        mesh=mesh,
        out_type=jax.ShapeDtypeStruct((L * T4, 8), jnp.float32),
        scratch_types=[
            pltpu.VMEM((PK_BLK,), jnp.float32),
            pltpu.VMEM((PK_BLK // 8, 8), jnp.float32),
            pltpu.SemaphoreType.DMA,
        ],
        compiler_params=_CP,
    )(_pack_body)
    return k(tbl_flat)


def _enc_body(x_hbm, tbl_hbm, out_hbm, xv, fxv, fyv, fzv, idxv, colv, rowsv,
              encv, sem0, sem1):
    wid = lax.axis_index("s") * NC + lax.axis_index("c")
    lane = lax.iota(jnp.int32, 16)
    sems = (sem0, sem1)

    def chunk_body(ci, carry):
        base = wid * PPW + ci * C
        for d in range(3):
            pltpu.sync_copy(x_hbm.at[pl.ds(d * N + base, C)], xv.at[d])

        def run_a(l, bf):
            res_half = float(RES[l]) * 0.5
            lROW = l * T4

            def group_a(g, carry_a):
                p0 = g * 16
                xr = xv[0, pl.ds(p0, 16)]
                yr = xv[1, pl.ds(p0, 16)]
                zr = xv[2, pl.ds(p0, 16)]
                px = (xr + 1.0) * res_half
                py = (yr + 1.0) * res_half
                pz = (zr + 1.0) * res_half
                ix = px.astype(jnp.int32)
                iy = py.astype(jnp.int32)
                iz = pz.astype(jnp.int32)
                fxv[bf, pl.ds(p0, 16)] = px - ix.astype(jnp.float32)
                fyv[bf, pl.ds(p0, 16)] = py - iy.astype(jnp.float32)
                fzv[bf, pl.ds(p0, 16)] = pz - iz.astype(jnp.float32)
                a0 = ix.astype(jnp.uint32)
                a1 = a0 + jnp.uint32(1)
                b0 = iy.astype(jnp.uint32) * P1
                b1 = b0 + P1
                c0 = iz.astype(jnp.uint32) * P2
                c1 = c0 + P2
                cc = 0
                for av in (a0, a1):
                    for bv in (b0, b1):
                        for cv in (c0, c1):
                            h = (av ^ bv ^ cv) & MASK
                            idxv[bf, pl.ds(cc * C + p0, 16)] = (
                                (h >> jnp.uint32(2)).astype(jnp.int32) + lROW)
                            colv[bf, pl.ds(cc * C + p0, 16)] = (
                                (h & jnp.uint32(3)).astype(jnp.int32) * 2)
                            cc += 1
                return carry_a

            lax.fori_loop(0, G, group_a, 0)
            return pltpu.async_copy(tbl_hbm.at[idxv.at[bf]], rowsv.at[bf],
                                    sems[bf])

        def run_b(l, bf):
            def group_b(g, carry_b):
                p0 = g * 16
                fx = fxv[bf, pl.ds(p0, 16)]
                fy = fyv[bf, pl.ds(p0, 16)]
                fz = fzv[bf, pl.ds(p0, 16)]
                cols = [colv[bf, pl.ds(c * C + p0, 16)] for c in range(8)]
                rows = [c * C + p0 + lane for c in range(8)]
                for f in (0, 1):
                    v = [plsc.load_gather(rowsv.at[bf], [rows[c], cols[c] + f])
                         for c in range(8)]
                    m00 = v[0] + fz * (v[1] - v[0])
                    m01 = v[2] + fz * (v[3] - v[2])
                    m10 = v[4] + fz * (v[5] - v[4])
                    m11 = v[6] + fz * (v[7] - v[6])
                    n0 = m00 + fy * (m01 - m00)
                    n1 = m10 + fy * (m11 - m10)
                    f2 = 2 * l + f
                    off = ((f2 // 8) * (8 * C) + (f2 % 8) * 128
                           + (p0 >> 7) * 1024 + (p0 & 127))
                    encv[pl.ds(off, 16)] = n0 + fx * (n1 - n0)
                return carry_b

            lax.fori_loop(0, G, group_b, 0)

        pending = run_a(0, 0)
        for l in range(1, L):
            nxt = run_a(l, l % 2)
            pending.wait()
            run_b(l - 1, (l - 1) % 2)
            pending = nxt
        pending.wait()
        run_b(L - 1, (L - 1) % 2)

        for ft in range(4):
            pltpu.sync_copy(
                encv.at[pl.ds(ft * (8 * C), 8 * C)],
                out_hbm.at[pl.ds(ft * (8 * N) + base * 8, 8 * C)])
        return carry

    lax.fori_loop(0, NCHUNK, chunk_body, 0)


def _encode_sc(x_flat, tbl_packed):
    mesh = plsc.VectorSubcoreMesh(core_axis_name="c", subcore_axis_name="s")
    k = functools.partial(
        pl.kernel,
        mesh=mesh,
        out_type=jax.ShapeDtypeStruct((2 * L * N,), jnp.float32),
        scratch_types=[
            pltpu.VMEM((3, C), jnp.float32),
            pltpu.VMEM((2, C), jnp.float32),
            pltpu.VMEM((2, C), jnp.float32),
            pltpu.VMEM((2, C), jnp.float32),
            pltpu.VMEM((2, 8 * C), jnp.int32),
            pltpu.VMEM((2, 8 * C), jnp.int32),
            pltpu.VMEM((2, 8 * C, 8), jnp.float32),
            pltpu.VMEM((2 * L * C,), jnp.float32),
            pltpu.SemaphoreType.DMA,
            pltpu.SemaphoreType.DMA,
        ],
        compiler_params=_CP,
    )(_enc_body)
    return k(x_flat, tbl_packed)


def _mlp_body(enc_ref, w1_ref, w2_ref, w3_ref, out_ref):
    hp = jax.lax.Precision.HIGHEST
    enc = enc_ref[...]
    h1 = jax.lax.dot_general(w1_ref[...], enc, (((0,), (0,)), ((), ())),
                             precision=hp, preferred_element_type=jnp.float32)
    h1 = jnp.maximum(h1, 0.0)
    h2 = jax.lax.dot_general(w2_ref[...], h1, (((0,), (0,)), ((), ())),
                             precision=hp, preferred_element_type=jnp.float32)
    h2 = jnp.maximum(h2, 0.0)
    out_ref[...] = jax.lax.dot_general(w3_ref[...], h2, (((0,), (0,)), ((), ())),
                                       precision=hp,
                                       preferred_element_type=jnp.float32)


def _mlp(enc_t, W1, W2, W3):
    n = enc_t.shape[1]
    bb = 8192
    grid = (n // bb,)
    return pl.pallas_call(
        _mlp_body,
        grid=grid,
        in_specs=[
            pl.BlockSpec((2 * L, bb), lambda i: (0, i)),
            pl.BlockSpec((32, 64), lambda i: (0, 0)),
            pl.BlockSpec((64, 64), lambda i: (0, 0)),
            pl.BlockSpec((64, 1), lambda i: (0, 0)),
        ],
        out_specs=pl.BlockSpec((1, bb), lambda i: (0, i)),
        out_shape=jax.ShapeDtypeStruct((1, n), jnp.float32),
    )(enc_t, W1, W2, W3)


def kernel(x, tables, W1, W2, W3):
    n = x.shape[0]
    x_flat = jnp.transpose(x).reshape(3 * n)
    tbl_flat = (tables.reshape(L, T // 128, 128, 2)
                .transpose(0, 1, 3, 2)
                .reshape(L * T * 2))
    tbl_packed = _pack_sc(tbl_flat)
    enc_flat = _encode_sc(x_flat, tbl_packed)
    enc_t = (enc_flat.reshape(4, n // 128, 8, 128)
             .transpose(0, 2, 1, 3)
             .reshape(2 * L, n))
    out_t = _mlp(enc_t, W1, W2, W3)
    return out_t.reshape(n, 1)
